# baseline (speedup 1.0000x reference)
import functools

import numpy as np
import jax
import jax.numpy as jnp
from jax import lax
from jax.experimental import pallas as pl
from jax.experimental.pallas import tpu as pltpu
from jax.experimental.pallas import tpu_sc as plsc

_B = 1024
_L = 200
_D = 128
_NC = 2
_NS = 16
_NW = _NC * _NS
_BW = _B // _NW
_HB = 4
_NH = _BW // _HB
_KO = 40
_NO = _L // _KO
_STEPS = _NO * _NH
_RPS = _HB * _KO
_ROWS_PER_W = _BW * _L
_LANES = 16
_NBUF = 4


def _sin_pe(seq_len, d_model):
    pos = np.arange(seq_len, dtype=np.float32)[:, None]
    div = np.exp(np.arange(0, d_model, 2, dtype=np.float32)
                 * (-np.log(10000.0) / d_model))
    ang = pos * div[None, :]
    pe = np.zeros((seq_len, d_model), dtype=np.float32)
    pe[:, 0::2] = np.sin(ang)
    pe[:, 1::2] = np.cos(ang)
    return pe


_PE = _sin_pe(_L, _D)


def _make_sc_kernel():
    mesh = plsc.VectorSubcoreMesh(core_axis_name="c", subcore_axis_name="s")

    @functools.partial(
        pl.kernel,
        mesh=mesh,
        out_type=(
            jax.ShapeDtypeStruct((_B, _L, _D), jnp.float32),
            jax.ShapeDtypeStruct((_B * _L,), jnp.int32),
        ),
        scratch_types=[
            pltpu.VMEM((_ROWS_PER_W,), jnp.int32),
            pltpu.VMEM((_NBUF, _RPS, _D), jnp.float32),
            pltpu.VMEM((_L, _D), jnp.float32),
            pltpu.VMEM((_ROWS_PER_W,), jnp.int32),
            pltpu.VMEM((_NBUF, _RPS), jnp.int32),
            pltpu.SemaphoreType.DMA((_NBUF,)),
            pltpu.SemaphoreType.DMA((_NBUF,)),
        ],
    )
    def emb_kernel(x_hbm, table_hbm, pe_hbm,
                   out_hbm, mask_hbm,
                   idx_v, rows_v, pe_v, mask_v, ioct_v, sg, ss):
        wid = lax.axis_index("s") * _NC + lax.axis_index("c")
        base = wid * _ROWS_PER_W
        wb0 = wid * _BW
        pltpu.sync_copy(x_hbm.at[pl.ds(base, _ROWS_PER_W)], idx_v)
        pltpu.sync_copy(pe_hbm, pe_v)

        def compact_idx(t, bu):
            o = t // _NH
            h = t % _NH
            lk = o * _KO
            for bi in range(_HB):
                s = (h * _HB + bi) * _L + lk
                d = bi * _KO
                for off in (0, 16, 24):
                    ioct_v[bu, pl.ds(d + off, _LANES)] = (
                        idx_v[pl.ds(s + off, _LANES)])

        def start_gather(t, bu):
            pltpu.async_copy(
                table_hbm.at[ioct_v.at[bu, pl.ds(0, 128)]],
                rows_v.at[bu, pl.ds(0, 128)], sg.at[bu])
            pltpu.async_copy(
                table_hbm.at[ioct_v.at[bu, pl.ds(128, _RPS - 128)]],
                rows_v.at[bu, pl.ds(128, _RPS - 128)], sg.at[bu])

        def wait_gather(bu):
            pltpu.make_async_copy(
                table_hbm.at[pl.ds(0, _RPS)],
                rows_v.at[bu], sg.at[bu]).wait()

        def start_scatter(t, bu):
            o = t // _NH
            h = t % _NH
            lk = o * _KO
            bb0 = wb0 + h * _HB
            for bi in range(_HB):
                pltpu.async_copy(
                    rows_v.at[bu, pl.ds(bi * _KO, _KO)],
                    out_hbm.at[bb0 + bi, pl.ds(lk, _KO)], ss.at[bu])

        def wait_scatter(bu):
            pltpu.make_async_copy(
                rows_v.at[bu], out_hbm.at[0, pl.ds(0, _RPS)], ss.at[bu]).wait()

        compact_idx(0, 0)
        start_gather(0, 0)
        compact_idx(1, 1)
        start_gather(1, 1)

        def step_body(t, carry):
            bu = t % _NBUF
            bn = (t + 2) % _NBUF
            o = t // _NH
            lk = o * _KO

            @pl.when(t >= _NBUF - 2)
            def _():
                wait_scatter(bn)

            @pl.when(t + 2 < _STEPS)
            def _():
                compact_idx(t + 2, bn)
                start_gather(t + 2, bn)

            wait_gather(bu)

            for lj in range(_KO):
                pe_regs = [pe_v[lk + lj, pl.ds(c * _LANES, _LANES)]
                           for c in range(_D // _LANES)]

                @plsc.parallel_loop(0, _HB, unroll=_HB)
                def add_body(bi):
                    r = bi * _KO + lj
                    for c in range(_D // _LANES):
                        plsc.addupdate(
                            rows_v.at[bu, r, pl.ds(c * _LANES, _LANES)],
                            pe_regs[c])

            start_scatter(t, bu)
            return carry

        lax.fori_loop(0, _STEPS, step_body, 0)

        def mask_body(i, carry):
            v = idx_v[pl.ds(i * _LANES, _LANES)]
            mask_v[pl.ds(i * _LANES, _LANES)] = jnp.where(
                v == 0, jnp.full((_LANES,), 1, jnp.int32),
                jnp.full((_LANES,), 0, jnp.int32))
            return carry

        lax.fori_loop(0, _ROWS_PER_W // _LANES, mask_body, 0)
        pltpu.sync_copy(mask_v, mask_hbm.at[pl.ds(base, _ROWS_PER_W)])

        for d in range(1, 3):
            wait_scatter((_STEPS - d) % _NBUF)

    return emb_kernel


_EMB_KERNEL = _make_sc_kernel()


def kernel(x, emb_table):
    x32 = x.astype(jnp.int32).reshape(_B * _L)
    pe = jnp.asarray(_PE)
    h, mask_i32 = _EMB_KERNEL(x32, emb_table, pe)
    padding_mask = mask_i32.reshape(_B, _L).astype(bool)
    return h, padding_mask


# speedup vs baseline: 1.1783x; 1.1783x over previous
import functools

import numpy as np
import jax
import jax.numpy as jnp
from jax import lax
from jax.experimental import pallas as pl
from jax.experimental.pallas import tpu as pltpu
from jax.experimental.pallas import tpu_sc as plsc

_B = 1024
_L = 200
_D = 128
_NC = 2
_NS = 16
_NW = _NC * _NS
_SEQ_PER_W = _B // _NW
_ROWS_PER_W = _SEQ_PER_W * _L
_LANES = 16
_NBUF = 3


def _sin_pe(seq_len, d_model):
    pos = np.arange(seq_len, dtype=np.float32)[:, None]
    div = np.exp(np.arange(0, d_model, 2, dtype=np.float32)
                 * (-np.log(10000.0) / d_model))
    ang = pos * div[None, :]
    pe = np.zeros((seq_len, d_model), dtype=np.float32)
    pe[:, 0::2] = np.sin(ang)
    pe[:, 1::2] = np.cos(ang)
    return pe


_PE = _sin_pe(_L, _D)


def _make_sc_kernel():
    mesh = plsc.VectorSubcoreMesh(core_axis_name="c", subcore_axis_name="s")

    @functools.partial(
        pl.kernel,
        mesh=mesh,
        out_type=(
            jax.ShapeDtypeStruct((_B * _L, _D), jnp.float32),
            jax.ShapeDtypeStruct((_B * _L,), jnp.int32),
        ),
        scratch_types=[
            pltpu.VMEM((_ROWS_PER_W,), jnp.int32),
            pltpu.VMEM((_NBUF, _L, _D), jnp.float32),
            pltpu.VMEM((_L, _D), jnp.float32),
            pltpu.VMEM((_ROWS_PER_W,), jnp.int32),
            pltpu.SemaphoreType.DMA((_NBUF,)),
            pltpu.SemaphoreType.DMA((_NBUF,)),
        ],
    )
    def emb_kernel(x_hbm, table_hbm, pe_hbm,
                   out_hbm, mask_hbm,
                   idx_v, rows_v, pe_v, mask_v, sg, ss):
        wid = lax.axis_index("s") * _NC + lax.axis_index("c")
        base = wid * _ROWS_PER_W

        pltpu.sync_copy(x_hbm.at[pl.ds(base, _ROWS_PER_W)], idx_v)
        pltpu.sync_copy(pe_hbm, pe_v)

        def start_gather(j, b):
            off = j * _L
            pltpu.async_copy(
                table_hbm.at[idx_v.at[pl.ds(off, 128)]],
                rows_v.at[b, pl.ds(0, 128)], sg.at[b])
            pltpu.async_copy(
                table_hbm.at[idx_v.at[pl.ds(off + 128, _L - 128)]],
                rows_v.at[b, pl.ds(128, _L - 128)], sg.at[b])

        def wait_gather(b):
            pltpu.make_async_copy(
                out_hbm.at[pl.ds(0, _L)], rows_v.at[b], sg.at[b]).wait()

        def start_scatter(j, b):
            pltpu.async_copy(
                rows_v.at[b], out_hbm.at[pl.ds(base + j * _L, _L)], ss.at[b])

        def wait_scatter(b):
            pltpu.make_async_copy(
                rows_v.at[b], out_hbm.at[pl.ds(0, _L)], ss.at[b]).wait()

        start_gather(0, 0)

        def seq_body(j, carry):
            b = j % _NBUF
            bn = (j + 1) % _NBUF

            @pl.when(j >= _NBUF - 1)
            def _():
                wait_scatter(bn)

            @pl.when(j + 1 < _SEQ_PER_W)
            def _():
                start_gather(j + 1, bn)

            wait_gather(b)

            @plsc.parallel_loop(0, _L, unroll=4)
            def add_body(r):
                for c in range(_D // _LANES):
                    sl = pl.ds(c * _LANES, _LANES)
                    plsc.addupdate(rows_v.at[b, r, sl], pe_v[r, sl])

            start_scatter(j, b)
            return carry

        lax.fori_loop(0, _SEQ_PER_W, seq_body, 0)

        def mask_body(i, carry):
            v = idx_v[pl.ds(i * _LANES, _LANES)]
            mask_v[pl.ds(i * _LANES, _LANES)] = jnp.where(
                v == 0, jnp.full((_LANES,), 1, jnp.int32),
                jnp.full((_LANES,), 0, jnp.int32))
            return carry

        lax.fori_loop(0, _ROWS_PER_W // _LANES, mask_body, 0)
        pltpu.sync_copy(mask_v, mask_hbm.at[pl.ds(base, _ROWS_PER_W)])

        wait_scatter((_SEQ_PER_W - 2) % _NBUF)
        wait_scatter((_SEQ_PER_W - 1) % _NBUF)

    return emb_kernel


_EMB_KERNEL = _make_sc_kernel()


def kernel(x, emb_table):
    x32 = x.astype(jnp.int32).reshape(_B * _L)
    pe = jnp.asarray(_PE)
    h_flat, mask_i32 = _EMB_KERNEL(x32, emb_table, pe)
    h = h_flat.reshape(_B, _L, _D)
    padding_mask = mask_i32.reshape(_B, _L).astype(bool)
    return h, padding_mask
